# Initial kernel scaffold; baseline (speedup 1.0000x reference)
import functools

import jax
import jax.numpy as jnp
from jax import lax
from jax.experimental import pallas as pl
from jax.experimental.pallas import tpu as pltpu
from jax.experimental.pallas import tpu_sc as plsc

F32 = jnp.float32


def _knn3_body(x1_ref, x2t_ref, idx_ref, w_ref, *, S, TN):
    b = pl.program_id(0)
    x1 = x1_ref[0]
    x2t = x2t_ref[0]
    x1sq = jnp.sum(x1 * x1, axis=1, keepdims=True)
    x2sq = jnp.sum(x2t * x2t, axis=0, keepdims=True)
    prod = jnp.dot(x1, x2t, preferred_element_type=F32)
    d = x1sq + x2sq - 2.0 * prod
    iota = lax.broadcasted_iota(jnp.int32, (TN, S), 1)
    inf = F32(jnp.inf)
    dm = d
    ms, ids = [], []
    for _ in range(3):
        m = jnp.min(dm, axis=1, keepdims=True)
        eq = dm == m
        fid = jnp.min(jnp.where(eq, iota, S), axis=1, keepdims=True)
        dm = jnp.where(iota == fid, inf, dm)
        ms.append(m)
        ids.append(fid)
    dists = jnp.concatenate(ms, axis=1)
    recip = 1.0 / (dists + 1e-8)
    w = recip / jnp.sum(recip, axis=1, keepdims=True)
    idx = jnp.concatenate(ids, axis=1) + b * S
    idx_ref[0] = idx
    w_ref[0] = w


def _knn3(x1p, x2tp, TN):
    B, N, _ = x1p.shape
    S = x2tp.shape[2]
    return pl.pallas_call(
        functools.partial(_knn3_body, S=S, TN=TN),
        grid=(B, N // TN),
        in_specs=[pl.BlockSpec((1, TN, 8), lambda b, i: (b, i, 0)),
                  pl.BlockSpec((1, 8, S), lambda b, i: (b, 0, 0))],
        out_specs=[pl.BlockSpec((1, TN, 3), lambda b, i: (b, i, 0)),
                   pl.BlockSpec((1, TN, 3), lambda b, i: (b, i, 0))],
        out_shape=[jax.ShapeDtypeStruct((B, N, 3), jnp.int32),
                   jax.ShapeDtypeStruct((B, N, 3), F32)],
    )(x1p, x2tp)


@functools.lru_cache(maxsize=None)
def _make_sc_interp(BN, QR, C):
    NW = 32
    R = BN // NW
    M = min(R, 32)
    NCH = R // M
    mesh = plsc.VectorSubcoreMesh(core_axis_name="c", subcore_axis_name="s")

    @functools.partial(
        pl.kernel,
        out_type=jax.ShapeDtypeStruct((BN, C), F32),
        mesh=mesh,
        scratch_types=[pltpu.VMEM((3 * M,), jnp.int32),
                       pltpu.VMEM((3 * M,), F32),
                       pltpu.VMEM((3 * M, C), F32),
                       pltpu.VMEM((M, C), F32),
                       pltpu.SemaphoreType.DMA],
    )
    def sc_interp(q_hbm, idx_hbm, w_hbm, out_hbm, idx_v, w_v, rows_v, out_v, sem):
        wid = lax.axis_index("s") * 2 + lax.axis_index("c")
        base = wid * R

        def chunk(ci, carry):
            rb = base + ci * M
            pltpu.sync_copy(idx_hbm.at[pl.ds(rb * 3, 3 * M)], idx_v)
            pltpu.sync_copy(w_hbm.at[pl.ds(rb * 3, 3 * M)], w_v)
            pltpu.async_copy(q_hbm.at[idx_v], rows_v, sem).wait()

            def row(r, c2):
                zi = jnp.zeros((16,), jnp.int32)
                w0 = plsc.load_gather(w_v, [zi + 3 * r])
                w1 = plsc.load_gather(w_v, [zi + (3 * r + 1)])
                w2 = plsc.load_gather(w_v, [zi + (3 * r + 2)])
                for j in range(C // 16):
                    sl = pl.ds(j * 16, 16)
                    out_v[r, sl] = (rows_v[3 * r, sl] * w0
                                    + rows_v[3 * r + 1, sl] * w1
                                    + rows_v[3 * r + 2, sl] * w2)
                return c2

            lax.fori_loop(0, M, row, 0)
            pltpu.sync_copy(out_v, out_hbm.at[pl.ds(rb, M)])
            return carry

        lax.fori_loop(0, NCH, chunk, 0)

    return sc_interp


def _sc_interp(q, idx_f, w_f):
    BN = idx_f.shape[0] // 3
    return _make_sc_interp(BN, q.shape[0], q.shape[1])(q, idx_f, w_f)


def _mm_body(x_ref, wT_ref, o_ref):
    o_ref[...] = jnp.dot(x_ref[...], wT_ref[...], preferred_element_type=F32)


def _mm(x, wT):
    BN = x.shape[0]
    C = wT.shape[1]
    return pl.pallas_call(
        _mm_body,
        out_shape=jax.ShapeDtypeStruct((BN, C), F32),
    )(x, wT)


def _mm_stats_body(x_ref, wT_ref, b_ref, add_ref, h_ref, s1_ref, s2_ref):
    h = (jnp.dot(x_ref[...], wT_ref[...], preferred_element_type=F32)
         + add_ref[...] + b_ref[...])
    h_ref[...] = h

    @pl.when(pl.program_id(0) == 0)
    def _():
        s1_ref[...] = jnp.zeros_like(s1_ref)
        s2_ref[...] = jnp.zeros_like(s2_ref)

    s1_ref[...] += jnp.sum(h, axis=0, keepdims=True)
    s2_ref[...] += jnp.sum(h * h, axis=0, keepdims=True)


def _mm_stats(x, wT, b, add, TM):
    BN, D = x.shape
    C = wT.shape[1]
    return pl.pallas_call(
        _mm_stats_body,
        grid=(BN // TM,),
        in_specs=[pl.BlockSpec((TM, D), lambda i: (i, 0)),
                  pl.BlockSpec((D, C), lambda i: (0, 0)),
                  pl.BlockSpec((1, C), lambda i: (0, 0)),
                  pl.BlockSpec((TM, C), lambda i: (i, 0))],
        out_specs=[pl.BlockSpec((TM, C), lambda i: (i, 0)),
                   pl.BlockSpec((1, C), lambda i: (0, 0)),
                   pl.BlockSpec((1, C), lambda i: (0, 0))],
        out_shape=[jax.ShapeDtypeStruct((BN, C), F32),
                   jax.ShapeDtypeStruct((1, C), F32),
                   jax.ShapeDtypeStruct((1, C), F32)],
    )(x, wT, b, add)


def _bn(h, s1, s2, g, be, count):
    mean = s1 * F32(1.0 / count)
    var = s2 * F32(1.0 / count) - mean * mean
    return jax.nn.relu((h - mean) * lax.rsqrt(var + 1e-5) * g + be)


def _bn_mm_stats_body(h_ref, s1_ref, s2_ref, g_ref, be_ref, wT_ref, b_ref,
                      h2_ref, t1_ref, t2_ref, *, count):
    xh = _bn(h_ref[...], s1_ref[...], s2_ref[...], g_ref[...], be_ref[...], count)
    h2 = jnp.dot(xh, wT_ref[...], preferred_element_type=F32) + b_ref[...]
    h2_ref[...] = h2

    @pl.when(pl.program_id(0) == 0)
    def _():
        t1_ref[...] = jnp.zeros_like(t1_ref)
        t2_ref[...] = jnp.zeros_like(t2_ref)

    t1_ref[...] += jnp.sum(h2, axis=0, keepdims=True)
    t2_ref[...] += jnp.sum(h2 * h2, axis=0, keepdims=True)


def _bn_mm_stats(h, s1, s2, g, be, wT, b, TM):
    BN, C1 = h.shape
    C2 = wT.shape[1]
    return pl.pallas_call(
        functools.partial(_bn_mm_stats_body, count=BN),
        grid=(BN // TM,),
        in_specs=[pl.BlockSpec((TM, C1), lambda i: (i, 0)),
                  pl.BlockSpec((1, C1), lambda i: (0, 0)),
                  pl.BlockSpec((1, C1), lambda i: (0, 0)),
                  pl.BlockSpec((1, C1), lambda i: (0, 0)),
                  pl.BlockSpec((1, C1), lambda i: (0, 0)),
                  pl.BlockSpec((C1, C2), lambda i: (0, 0)),
                  pl.BlockSpec((1, C2), lambda i: (0, 0))],
        out_specs=[pl.BlockSpec((TM, C2), lambda i: (i, 0)),
                   pl.BlockSpec((1, C2), lambda i: (0, 0)),
                   pl.BlockSpec((1, C2), lambda i: (0, 0))],
        out_shape=[jax.ShapeDtypeStruct((BN, C2), F32),
                   jax.ShapeDtypeStruct((1, C2), F32),
                   jax.ShapeDtypeStruct((1, C2), F32)],
    )(h, s1, s2, g, be, wT, b)


def _bn_q_body(h_ref, s1_ref, s2_ref, g_ref, be_ref, wT_ref, q_ref, *, count):
    xh = _bn(h_ref[...], s1_ref[...], s2_ref[...], g_ref[...], be_ref[...], count)
    q_ref[...] = jnp.dot(xh, wT_ref[...], preferred_element_type=F32)


def _bn_q(h, s1, s2, g, be, wT, TM):
    BN, C = h.shape
    CQ = wT.shape[1]
    return pl.pallas_call(
        functools.partial(_bn_q_body, count=BN),
        grid=(BN // TM,),
        in_specs=[pl.BlockSpec((TM, C), lambda i: (i, 0)),
                  pl.BlockSpec((1, C), lambda i: (0, 0)),
                  pl.BlockSpec((1, C), lambda i: (0, 0)),
                  pl.BlockSpec((1, C), lambda i: (0, 0)),
                  pl.BlockSpec((1, C), lambda i: (0, 0)),
                  pl.BlockSpec((C, CQ), lambda i: (0, 0))],
        out_specs=pl.BlockSpec((TM, CQ), lambda i: (i, 0)),
        out_shape=jax.ShapeDtypeStruct((BN, CQ), F32),
    )(h, s1, s2, g, be, wT)


def _bn_out_body(h_ref, s1_ref, s2_ref, g_ref, be_ref, o_ref, *, count):
    o_ref[...] = _bn(h_ref[...], s1_ref[...], s2_ref[...], g_ref[...],
                     be_ref[...], count)


def _bn_out(h, s1, s2, g, be, TM):
    BN, C = h.shape
    return pl.pallas_call(
        functools.partial(_bn_out_body, count=BN),
        grid=(BN // TM,),
        in_specs=[pl.BlockSpec((TM, C), lambda i: (i, 0)),
                  pl.BlockSpec((1, C), lambda i: (0, 0)),
                  pl.BlockSpec((1, C), lambda i: (0, 0)),
                  pl.BlockSpec((1, C), lambda i: (0, 0)),
                  pl.BlockSpec((1, C), lambda i: (0, 0))],
        out_specs=pl.BlockSpec((TM, C), lambda i: (i, 0)),
        out_shape=jax.ShapeDtypeStruct((BN, C), F32),
    )(h, s1, s2, g, be)


def _row2(v):
    return v.reshape(1, -1)


def _stage(x1p, x2tp, p1pm, q, W1aT, b1, g1, be1, W2T, b2, g2, be2,
           TN, TM, W1bT_next=None):
    B, N, _ = x1p.shape
    BN = B * N
    idx, w = _knn3(x1p, x2tp, TN)
    interp = _sc_interp(q, idx.reshape(BN * 3), w.reshape(BN * 3))
    h1, s1, s2 = _mm_stats(p1pm, W1aT, _row2(b1), interp, TM)
    h2, t1, t2 = _bn_mm_stats(h1, s1, s2, _row2(g1), _row2(be1), W2T,
                              _row2(b2), TM)
    if W1bT_next is None:
        return _bn_out(h2, t1, t2, _row2(g2), _row2(be2), TM)
    return _bn_q(h2, t1, t2, _row2(g2), _row2(be2), W1bT_next, TM)


def _prep_x1(p):
    return jnp.pad(jnp.transpose(p, (0, 2, 1)), ((0, 0), (0, 0), (0, 5)))


def _prep_x2(p):
    return jnp.pad(p, ((0, 0), (0, 5), (0, 0)))


def _pm(feats):
    B, C, N = feats.shape
    return jnp.transpose(feats, (0, 2, 1)).reshape(B * N, C)


def kernel(pts0, pts1, pts2, pts3, feats0, feats1, feats2, feats3,
           fp3_W1, fp3_b1, fp3_g1, fp3_be1, fp3_W2, fp3_b2, fp3_g2, fp3_be2,
           fp2_W1, fp2_b1, fp2_g1, fp2_be1, fp2_W2, fp2_b2, fp2_g2, fp2_be2,
           fp1_W1, fp1_b1, fp1_g1, fp1_be1, fp1_W2, fp1_b2, fp1_g2, fp1_be2):
    B = pts0.shape[0]
    N0 = pts0.shape[2]

    q3 = _mm(_pm(feats3), fp3_W1[:, 512:].T)

    q2 = _stage(_prep_x1(pts2), _prep_x2(pts3), _pm(feats2), q3,
                fp3_W1[:, :512].T, fp3_b1, fp3_g1, fp3_be1,
                fp3_W2.T, fp3_b2, fp3_g2, fp3_be2,
                TN=256, TM=1024, W1bT_next=fp2_W1[:, 320:].T)

    q1 = _stage(_prep_x1(pts1), _prep_x2(pts2), _pm(feats1), q2,
                fp2_W1[:, :320].T, fp2_b1, fp2_g1, fp2_be1,
                fp2_W2.T, fp2_b2, fp2_g2, fp2_be2,
                TN=512, TM=2048, W1bT_next=fp1_W1[:, 6:].T)

    p1pm1 = jnp.pad(_pm(jnp.concatenate([pts0, feats0], axis=1)),
                    ((0, 0), (0, 2)))
    W1aT1 = jnp.pad(fp1_W1[:, :6].T, ((0, 2), (0, 0)))
    out = _stage(_prep_x1(pts0), _prep_x2(pts1), p1pm1, q1,
                 W1aT1, fp1_b1, fp1_g1, fp1_be1,
                 fp1_W2.T, fp1_b2, fp1_g2, fp1_be2,
                 TN=512, TM=2048, W1bT_next=None)

    return jnp.transpose(out.reshape(B, N0, -1), (0, 2, 1))


# speedup vs baseline: 9.0766x; 9.0766x over previous
import functools

import jax
import jax.numpy as jnp
from jax import lax
from jax.experimental import pallas as pl
from jax.experimental.pallas import tpu as pltpu
from jax.experimental.pallas import tpu_sc as plsc

F32 = jnp.float32


def _knn3_body(x1_ref, x2t_ref, idx_ref, w_ref, *, S, TN):
    b = pl.program_id(0)
    x1 = x1_ref[0]
    x2t = x2t_ref[0]
    x1sq = ((x1[:, 0:1] * x1[:, 0:1] + x1[:, 1:2] * x1[:, 1:2])
            + x1[:, 2:3] * x1[:, 2:3])
    x2sq = ((x2t[0:1, :] * x2t[0:1, :] + x2t[1:2, :] * x2t[1:2, :])
            + x2t[2:3, :] * x2t[2:3, :])
    x1b = x1.astype(jnp.bfloat16).astype(F32)
    x2b = x2t.astype(jnp.bfloat16).astype(F32)
    sh = (TN, S)
    p0 = jnp.broadcast_to(x1b[:, 0:1], sh) * jnp.broadcast_to(x2b[0:1, :], sh)
    p1 = jnp.broadcast_to(x1b[:, 1:2], sh) * jnp.broadcast_to(x2b[1:2, :], sh)
    p2 = jnp.broadcast_to(x1b[:, 2:3], sh) * jnp.broadcast_to(x2b[2:3, :], sh)
    t = p1 + p2
    bp = t - p1
    e1 = (p1 - (t - bp)) + (p2 - bp)
    s = p0 + t
    bp2 = s - p0
    e2 = (p0 - (s - bp2)) + (t - bp2)
    prod = s + (e1 + e2)
    d = (x1sq + x2sq) - 2.0 * prod
    iota = lax.broadcasted_iota(jnp.int32, (TN, S), 1)
    inf = F32(jnp.inf)
    dm = d
    ms, ids = [], []
    for _ in range(3):
        m = jnp.min(dm, axis=1, keepdims=True)
        eq = dm == m
        fid = jnp.min(jnp.where(eq, iota, S), axis=1, keepdims=True)
        dm = jnp.where(iota == fid, inf, dm)
        ms.append(m)
        ids.append(fid)
    dists = jnp.concatenate(ms, axis=1)
    recip = 1.0 / (dists + 1e-8)
    w = recip / jnp.sum(recip, axis=1, keepdims=True)
    idx = jnp.concatenate(ids, axis=1) + b * S
    idx_ref[0] = idx
    w_ref[0] = w


def _knn3(x1p, x2tp, TN):
    B, N, _ = x1p.shape
    S = x2tp.shape[2]
    return pl.pallas_call(
        functools.partial(_knn3_body, S=S, TN=TN),
        grid=(B, N // TN),
        in_specs=[pl.BlockSpec((1, TN, 8), lambda b, i: (b, i, 0)),
                  pl.BlockSpec((1, 8, S), lambda b, i: (b, 0, 0))],
        out_specs=[pl.BlockSpec((1, TN, 3), lambda b, i: (b, i, 0)),
                   pl.BlockSpec((1, TN, 3), lambda b, i: (b, i, 0))],
        out_shape=[jax.ShapeDtypeStruct((B, N, 3), jnp.int32),
                   jax.ShapeDtypeStruct((B, N, 3), F32)],
    )(x1p, x2tp)


@functools.lru_cache(maxsize=None)
def _make_sc_interp(BN, QR, C):
    NW = 32
    R = BN // NW
    M = min(R, 32)
    NCH = R // M
    mesh = plsc.VectorSubcoreMesh(core_axis_name="c", subcore_axis_name="s")

    @functools.partial(
        pl.kernel,
        out_type=jax.ShapeDtypeStruct((BN, C), F32),
        mesh=mesh,
        scratch_types=[pltpu.VMEM((3 * M,), jnp.int32),
                       pltpu.VMEM((3 * M + 16,), F32),
                       pltpu.VMEM((3 * M, C), F32),
                       pltpu.VMEM((M, C), F32),
                       pltpu.SemaphoreType.DMA],
    )
    def sc_interp(q_hbm, idx_hbm, w_hbm, out_hbm, idx_v, w_v, rows_v, out_v, sem):
        wid = lax.axis_index("s") * 2 + lax.axis_index("c")
        base = wid * R

        def chunk(ci, carry):
            rb = base + ci * M
            pltpu.sync_copy(idx_hbm.at[pl.ds(rb * 3, 3 * M)], idx_v)
            pltpu.sync_copy(w_hbm.at[pl.ds(rb * 3, 3 * M)],
                            w_v.at[pl.ds(0, 3 * M)])
            pltpu.async_copy(q_hbm.at[idx_v], rows_v, sem).wait()

            def row(r, c2):
                wv3 = w_v[pl.ds(3 * r, 16)]
                w0 = wv3[0]
                w1 = wv3[1]
                w2 = wv3[2]
                for j in range(C // 16):
                    sl = pl.ds(j * 16, 16)
                    out_v[r, sl] = (rows_v[3 * r, sl] * w0
                                    + rows_v[3 * r + 1, sl] * w1
                                    + rows_v[3 * r + 2, sl] * w2)
                return c2

            lax.fori_loop(0, M, row, 0)
            pltpu.sync_copy(out_v, out_hbm.at[pl.ds(rb, M)])
            return carry

        lax.fori_loop(0, NCH, chunk, 0)

    return sc_interp


def _sc_interp(q, idx_f, w_f):
    BN = idx_f.shape[0] // 3
    return _make_sc_interp(BN, q.shape[0], q.shape[1])(q, idx_f, w_f)


def _mm_stats_body(x_ref, wT_ref, i_ref, wbT_ref, b_ref, h_ref, s1_ref, s2_ref):
    h = (jnp.dot(x_ref[...], wT_ref[...], preferred_element_type=F32)
         + jnp.dot(i_ref[...], wbT_ref[...], preferred_element_type=F32)
         + b_ref[...])
    h_ref[...] = h

    @pl.when(pl.program_id(0) == 0)
    def _():
        s1_ref[...] = jnp.zeros_like(s1_ref)
        s2_ref[...] = jnp.zeros_like(s2_ref)

    s1_ref[...] += jnp.sum(h, axis=0, keepdims=True)
    s2_ref[...] += jnp.sum(h * h, axis=0, keepdims=True)


def _mm_stats(x, wT, interp, wbT, b, TM):
    BN, D = x.shape
    D2 = wbT.shape[0]
    C = wT.shape[1]
    return pl.pallas_call(
        _mm_stats_body,
        grid=(BN // TM,),
        in_specs=[pl.BlockSpec((TM, D), lambda i: (i, 0)),
                  pl.BlockSpec((D, C), lambda i: (0, 0)),
                  pl.BlockSpec((TM, D2), lambda i: (i, 0)),
                  pl.BlockSpec((D2, C), lambda i: (0, 0)),
                  pl.BlockSpec((1, C), lambda i: (0, 0))],
        out_specs=[pl.BlockSpec((TM, C), lambda i: (i, 0)),
                   pl.BlockSpec((1, C), lambda i: (0, 0)),
                   pl.BlockSpec((1, C), lambda i: (0, 0))],
        out_shape=[jax.ShapeDtypeStruct((BN, C), F32),
                   jax.ShapeDtypeStruct((1, C), F32),
                   jax.ShapeDtypeStruct((1, C), F32)],
    )(x, wT, interp, wbT, b)


def _bn(h, s1, s2, g, be, count):
    mean = s1 * F32(1.0 / count)
    var = s2 * F32(1.0 / count) - mean * mean
    return jax.nn.relu((h - mean) * lax.rsqrt(var + 1e-5) * g + be)


def _bn_mm_stats_body(h_ref, s1_ref, s2_ref, g_ref, be_ref, wT_ref, b_ref,
                      h2_ref, t1_ref, t2_ref, *, count):
    xh = _bn(h_ref[...], s1_ref[...], s2_ref[...], g_ref[...], be_ref[...], count)
    h2 = jnp.dot(xh, wT_ref[...], preferred_element_type=F32) + b_ref[...]
    h2_ref[...] = h2

    @pl.when(pl.program_id(0) == 0)
    def _():
        t1_ref[...] = jnp.zeros_like(t1_ref)
        t2_ref[...] = jnp.zeros_like(t2_ref)

    t1_ref[...] += jnp.sum(h2, axis=0, keepdims=True)
    t2_ref[...] += jnp.sum(h2 * h2, axis=0, keepdims=True)


def _bn_mm_stats(h, s1, s2, g, be, wT, b, TM):
    BN, C1 = h.shape
    C2 = wT.shape[1]
    return pl.pallas_call(
        functools.partial(_bn_mm_stats_body, count=BN),
        grid=(BN // TM,),
        in_specs=[pl.BlockSpec((TM, C1), lambda i: (i, 0)),
                  pl.BlockSpec((1, C1), lambda i: (0, 0)),
                  pl.BlockSpec((1, C1), lambda i: (0, 0)),
                  pl.BlockSpec((1, C1), lambda i: (0, 0)),
                  pl.BlockSpec((1, C1), lambda i: (0, 0)),
                  pl.BlockSpec((C1, C2), lambda i: (0, 0)),
                  pl.BlockSpec((1, C2), lambda i: (0, 0))],
        out_specs=[pl.BlockSpec((TM, C2), lambda i: (i, 0)),
                   pl.BlockSpec((1, C2), lambda i: (0, 0)),
                   pl.BlockSpec((1, C2), lambda i: (0, 0))],
        out_shape=[jax.ShapeDtypeStruct((BN, C2), F32),
                   jax.ShapeDtypeStruct((1, C2), F32),
                   jax.ShapeDtypeStruct((1, C2), F32)],
    )(h, s1, s2, g, be, wT, b)


def _bn_out_body(h_ref, s1_ref, s2_ref, g_ref, be_ref, o_ref, *, count):
    o_ref[...] = _bn(h_ref[...], s1_ref[...], s2_ref[...], g_ref[...],
                     be_ref[...], count)


def _bn_out(h, s1, s2, g, be, TM):
    BN, C = h.shape
    return pl.pallas_call(
        functools.partial(_bn_out_body, count=BN),
        grid=(BN // TM,),
        in_specs=[pl.BlockSpec((TM, C), lambda i: (i, 0)),
                  pl.BlockSpec((1, C), lambda i: (0, 0)),
                  pl.BlockSpec((1, C), lambda i: (0, 0)),
                  pl.BlockSpec((1, C), lambda i: (0, 0)),
                  pl.BlockSpec((1, C), lambda i: (0, 0))],
        out_specs=pl.BlockSpec((TM, C), lambda i: (i, 0)),
        out_shape=jax.ShapeDtypeStruct((BN, C), F32),
    )(h, s1, s2, g, be)


def _row2(v):
    return v.reshape(1, -1)


def _stage(x1p, x2tp, p1pm, src, W1aT, W1bT, b1, g1, be1, W2T, b2, g2, be2,
           TN, TM):
    B, N, _ = x1p.shape
    BN = B * N
    idx, w = _knn3(x1p, x2tp, TN)
    interp = _sc_interp(src, idx.reshape(BN * 3), w.reshape(BN * 3))
    h1, s1, s2 = _mm_stats(p1pm, W1aT, interp, W1bT, _row2(b1), TM)
    h2, t1, t2 = _bn_mm_stats(h1, s1, s2, _row2(g1), _row2(be1), W2T,
                              _row2(b2), TM)
    return _bn_out(h2, t1, t2, _row2(g2), _row2(be2), TM)


def _prep_x1(p):
    return jnp.pad(jnp.transpose(p, (0, 2, 1)), ((0, 0), (0, 0), (0, 5)))


def _prep_x2(p):
    return jnp.pad(p, ((0, 0), (0, 5), (0, 0)))


def _pm(feats):
    B, C, N = feats.shape
    return jnp.transpose(feats, (0, 2, 1)).reshape(B * N, C)


def kernel(pts0, pts1, pts2, pts3, feats0, feats1, feats2, feats3,
           fp3_W1, fp3_b1, fp3_g1, fp3_be1, fp3_W2, fp3_b2, fp3_g2, fp3_be2,
           fp2_W1, fp2_b1, fp2_g1, fp2_be1, fp2_W2, fp2_b2, fp2_g2, fp2_be2,
           fp1_W1, fp1_b1, fp1_g1, fp1_be1, fp1_W2, fp1_b2, fp1_g2, fp1_be2):
    B = pts0.shape[0]
    N0 = pts0.shape[2]

    l2 = _stage(_prep_x1(pts2), _prep_x2(pts3), _pm(feats2), _pm(feats3),
                fp3_W1[:, :512].T, fp3_W1[:, 512:].T, fp3_b1, fp3_g1, fp3_be1,
                fp3_W2.T, fp3_b2, fp3_g2, fp3_be2, TN=256, TM=1024)

    l1 = _stage(_prep_x1(pts1), _prep_x2(pts2), _pm(feats1), l2,
                fp2_W1[:, :320].T, fp2_W1[:, 320:].T, fp2_b1, fp2_g1, fp2_be1,
                fp2_W2.T, fp2_b2, fp2_g2, fp2_be2, TN=512, TM=2048)

    p1pm1 = jnp.pad(_pm(jnp.concatenate([pts0, feats0], axis=1)),
                    ((0, 0), (0, 2)))
    W1aT1 = jnp.pad(fp1_W1[:, :6].T, ((0, 2), (0, 0)))
    out = _stage(_prep_x1(pts0), _prep_x2(pts1), p1pm1, l1,
                 W1aT1, fp1_W1[:, 6:].T, fp1_b1, fp1_g1, fp1_be1,
                 fp1_W2.T, fp1_b2, fp1_g2, fp1_be2, TN=512, TM=2048)

    return jnp.transpose(out.reshape(B, N0, -1), (0, 2, 1))


# speedup vs baseline: 9.9618x; 1.0975x over previous
import functools

import jax
import jax.numpy as jnp
from jax import lax
from jax.experimental import pallas as pl
from jax.experimental.pallas import tpu as pltpu
from jax.experimental.pallas import tpu_sc as plsc

F32 = jnp.float32


def _knn3_body(x1_ref, x2t_ref, idx_ref, w_ref, *, S, TN):
    b = pl.program_id(0)
    x1 = x1_ref[0]
    x2t = x2t_ref[0]
    x1sq = ((x1[:, 0:1] * x1[:, 0:1] + x1[:, 1:2] * x1[:, 1:2])
            + x1[:, 2:3] * x1[:, 2:3])
    x2sq = ((x2t[0:1, :] * x2t[0:1, :] + x2t[1:2, :] * x2t[1:2, :])
            + x2t[2:3, :] * x2t[2:3, :])
    x1b = x1.astype(jnp.bfloat16).astype(F32)
    x2b = x2t.astype(jnp.bfloat16).astype(F32)
    sh = (TN, S)
    p0 = jnp.broadcast_to(x1b[:, 0:1], sh) * jnp.broadcast_to(x2b[0:1, :], sh)
    p1 = jnp.broadcast_to(x1b[:, 1:2], sh) * jnp.broadcast_to(x2b[1:2, :], sh)
    p2 = jnp.broadcast_to(x1b[:, 2:3], sh) * jnp.broadcast_to(x2b[2:3, :], sh)
    t = p1 + p2
    bp = t - p1
    e1 = (p1 - (t - bp)) + (p2 - bp)
    s = p0 + t
    bp2 = s - p0
    e2 = (p0 - (s - bp2)) + (t - bp2)
    prod = s + (e1 + e2)
    d = (x1sq + x2sq) - 2.0 * prod
    iota = lax.broadcasted_iota(jnp.int32, (TN, S), 1)
    inf = F32(jnp.inf)
    dm = d
    ms, ids = [], []
    for _ in range(3):
        m = jnp.min(dm, axis=1, keepdims=True)
        eq = dm == m
        fid = jnp.min(jnp.where(eq, iota, S), axis=1, keepdims=True)
        dm = jnp.where(iota == fid, inf, dm)
        ms.append(m)
        ids.append(fid)
    dists = jnp.concatenate(ms, axis=1)
    recip = 1.0 / (dists + 1e-8)
    w = recip / jnp.sum(recip, axis=1, keepdims=True)
    idx = jnp.concatenate(ids, axis=1) + b * S
    idx_ref[0] = idx
    w_ref[0] = w


def _knn3(x1p, x2tp, TN):
    B, N, _ = x1p.shape
    S = x2tp.shape[2]
    return pl.pallas_call(
        functools.partial(_knn3_body, S=S, TN=TN),
        grid=(B, N // TN),
        in_specs=[pl.BlockSpec((1, TN, 8), lambda b, i: (b, i, 0)),
                  pl.BlockSpec((1, 8, S), lambda b, i: (b, 0, 0))],
        out_specs=[pl.BlockSpec((1, TN, 3), lambda b, i: (b, i, 0)),
                   pl.BlockSpec((1, TN, 3), lambda b, i: (b, i, 0))],
        out_shape=[jax.ShapeDtypeStruct((B, N, 3), jnp.int32),
                   jax.ShapeDtypeStruct((B, N, 3), F32)],
    )(x1p, x2tp)


@functools.lru_cache(maxsize=None)
def _make_sc_interp(BN, QR, C):
    NW = 32
    R = BN // NW
    M = min(R, 32)
    NCH = R // M
    mesh = plsc.VectorSubcoreMesh(core_axis_name="c", subcore_axis_name="s")

    def compute_chunk(ci, w_v, rows_vp, out_vp):
        def row(r, c2):
            wv3 = w_v[pl.ds(3 * (ci * M + r), 16)]
            w0 = wv3[0]
            w1 = wv3[1]
            w2 = wv3[2]
            for j in range(C // 16):
                sl = pl.ds(j * 16, 16)
                out_vp[r, sl] = (rows_vp[3 * r, sl] * w0
                                 + rows_vp[3 * r + 1, sl] * w1
                                 + rows_vp[3 * r + 2, sl] * w2)
            return c2

        lax.fori_loop(0, M, row, 0)

    if NCH == 1:
        @functools.partial(
            pl.kernel,
            out_type=jax.ShapeDtypeStruct((BN, C), F32),
            mesh=mesh,
            scratch_types=[pltpu.VMEM((3 * M,), jnp.int32),
                           pltpu.VMEM((3 * M + 16,), F32),
                           pltpu.VMEM((3 * M, C), F32),
                           pltpu.VMEM((M, C), F32),
                           pltpu.SemaphoreType.DMA],
        )
        def sc_interp(q_hbm, idx_hbm, w_hbm, out_hbm, idx_v, w_v, rows_v,
                      out_v, sem):
            wid = lax.axis_index("s") * 2 + lax.axis_index("c")
            base = wid * R
            pltpu.sync_copy(idx_hbm.at[wid], idx_v)
            pltpu.sync_copy(w_hbm.at[pl.ds(base * 3, 3 * M)],
                            w_v.at[pl.ds(0, 3 * M)])
            pltpu.async_copy(q_hbm.at[idx_v], rows_v, sem).wait()
            compute_chunk(0, w_v, rows_v, out_v)
            pltpu.sync_copy(out_v, out_hbm.at[pl.ds(base, M)])

        return sc_interp

    assert NCH % 2 == 0

    @functools.partial(
        pl.kernel,
        out_type=jax.ShapeDtypeStruct((BN, C), F32),
        mesh=mesh,
        scratch_types=[pltpu.VMEM((NCH, 3 * M), jnp.int32),
                       pltpu.VMEM((3 * R + 16,), F32),
                       pltpu.VMEM((2, 3 * M, C), F32),
                       pltpu.VMEM((2, M, C), F32),
                       pltpu.SemaphoreType.DMA,
                       pltpu.SemaphoreType.DMA,
                       pltpu.SemaphoreType.DMA,
                       pltpu.SemaphoreType.DMA],
    )
    def sc_interp(q_hbm, idx_hbm, w_hbm, out_hbm, idx_v, w_v, rows_v, out_v,
                  gsem0, gsem1, osem0, osem1):
        wid = lax.axis_index("s") * 2 + lax.axis_index("c")
        base = wid * R
        pltpu.sync_copy(idx_hbm.at[wid], idx_v)
        pltpu.sync_copy(w_hbm.at[pl.ds(base * 3, 3 * R)],
                        w_v.at[pl.ds(0, 3 * R)])
        pltpu.async_copy(q_hbm.at[idx_v.at[0]], rows_v.at[0], gsem0)

        def body(k, carry):
            ci0 = 2 * k
            ci1 = ci0 + 1
            pltpu.async_copy(q_hbm.at[idx_v.at[ci1]], rows_v.at[1], gsem1)
            pltpu.make_async_copy(q_hbm.at[idx_v.at[0]], rows_v.at[0],
                                  gsem0).wait()

            @pl.when(k > 0)
            def _():
                pltpu.make_async_copy(out_v.at[0],
                                      out_hbm.at[pl.ds(base, M)],
                                      osem0).wait()

            compute_chunk(ci0, w_v, rows_v.at[0], out_v.at[0])
            pltpu.async_copy(out_v.at[0], out_hbm.at[pl.ds(base + ci0 * M, M)],
                             osem0)

            @pl.when(ci1 + 1 < NCH)
            def _():
                pltpu.async_copy(q_hbm.at[idx_v.at[ci1 + 1]], rows_v.at[0],
                                 gsem0)

            pltpu.make_async_copy(q_hbm.at[idx_v.at[0]], rows_v.at[1],
                                  gsem1).wait()

            @pl.when(k > 0)
            def _():
                pltpu.make_async_copy(out_v.at[1],
                                      out_hbm.at[pl.ds(base, M)],
                                      osem1).wait()

            compute_chunk(ci1, w_v, rows_v.at[1], out_v.at[1])
            pltpu.async_copy(out_v.at[1], out_hbm.at[pl.ds(base + ci1 * M, M)],
                             osem1)
            return carry

        lax.fori_loop(0, NCH // 2, body, 0)
        pltpu.make_async_copy(out_v.at[0], out_hbm.at[pl.ds(base, M)],
                              osem0).wait()
        pltpu.make_async_copy(out_v.at[1], out_hbm.at[pl.ds(base, M)],
                              osem1).wait()

    return sc_interp


def _sc_interp(q, idx_f, w_f):
    BN = idx_f.shape[0] // 3
    NW = 32
    R = BN // NW
    M = min(R, 32)
    NCH = R // M
    if NCH == 1:
        idx3 = idx_f.reshape(NW, 3 * M)
    else:
        idx3 = idx_f.reshape(NW, NCH, 3 * M)
    return _make_sc_interp(BN, q.shape[0], q.shape[1])(q, idx3, w_f)


def _mm_stats_body(x_ref, wT_ref, i_ref, wbT_ref, b_ref, h_ref, s1_ref, s2_ref):
    h = (jnp.dot(x_ref[...], wT_ref[...], preferred_element_type=F32)
         + jnp.dot(i_ref[...], wbT_ref[...], preferred_element_type=F32)
         + b_ref[...])
    h_ref[...] = h

    @pl.when(pl.program_id(0) == 0)
    def _():
        s1_ref[...] = jnp.zeros_like(s1_ref)
        s2_ref[...] = jnp.zeros_like(s2_ref)

    s1_ref[...] += jnp.sum(h, axis=0, keepdims=True)
    s2_ref[...] += jnp.sum(h * h, axis=0, keepdims=True)


def _mm_stats(x, wT, interp, wbT, b, TM):
    BN, D = x.shape
    D2 = wbT.shape[0]
    C = wT.shape[1]
    return pl.pallas_call(
        _mm_stats_body,
        grid=(BN // TM,),
        in_specs=[pl.BlockSpec((TM, D), lambda i: (i, 0)),
                  pl.BlockSpec((D, C), lambda i: (0, 0)),
                  pl.BlockSpec((TM, D2), lambda i: (i, 0)),
                  pl.BlockSpec((D2, C), lambda i: (0, 0)),
                  pl.BlockSpec((1, C), lambda i: (0, 0))],
        out_specs=[pl.BlockSpec((TM, C), lambda i: (i, 0)),
                   pl.BlockSpec((1, C), lambda i: (0, 0)),
                   pl.BlockSpec((1, C), lambda i: (0, 0))],
        out_shape=[jax.ShapeDtypeStruct((BN, C), F32),
                   jax.ShapeDtypeStruct((1, C), F32),
                   jax.ShapeDtypeStruct((1, C), F32)],
    )(x, wT, interp, wbT, b)


def _bn(h, s1, s2, g, be, count):
    mean = s1 * F32(1.0 / count)
    var = s2 * F32(1.0 / count) - mean * mean
    return jax.nn.relu((h - mean) * lax.rsqrt(var + 1e-5) * g + be)


def _bn_mm_stats_body(h_ref, s1_ref, s2_ref, g_ref, be_ref, wT_ref, b_ref,
                      h2_ref, t1_ref, t2_ref, *, count):
    xh = _bn(h_ref[...], s1_ref[...], s2_ref[...], g_ref[...], be_ref[...], count)
    h2 = jnp.dot(xh, wT_ref[...], preferred_element_type=F32) + b_ref[...]
    h2_ref[...] = h2

    @pl.when(pl.program_id(0) == 0)
    def _():
        t1_ref[...] = jnp.zeros_like(t1_ref)
        t2_ref[...] = jnp.zeros_like(t2_ref)

    t1_ref[...] += jnp.sum(h2, axis=0, keepdims=True)
    t2_ref[...] += jnp.sum(h2 * h2, axis=0, keepdims=True)


def _bn_mm_stats(h, s1, s2, g, be, wT, b, TM):
    BN, C1 = h.shape
    C2 = wT.shape[1]
    return pl.pallas_call(
        functools.partial(_bn_mm_stats_body, count=BN),
        grid=(BN // TM,),
        in_specs=[pl.BlockSpec((TM, C1), lambda i: (i, 0)),
                  pl.BlockSpec((1, C1), lambda i: (0, 0)),
                  pl.BlockSpec((1, C1), lambda i: (0, 0)),
                  pl.BlockSpec((1, C1), lambda i: (0, 0)),
                  pl.BlockSpec((1, C1), lambda i: (0, 0)),
                  pl.BlockSpec((C1, C2), lambda i: (0, 0)),
                  pl.BlockSpec((1, C2), lambda i: (0, 0))],
        out_specs=[pl.BlockSpec((TM, C2), lambda i: (i, 0)),
                   pl.BlockSpec((1, C2), lambda i: (0, 0)),
                   pl.BlockSpec((1, C2), lambda i: (0, 0))],
        out_shape=[jax.ShapeDtypeStruct((BN, C2), F32),
                   jax.ShapeDtypeStruct((1, C2), F32),
                   jax.ShapeDtypeStruct((1, C2), F32)],
    )(h, s1, s2, g, be, wT, b)


def _bn_out_body(h_ref, s1_ref, s2_ref, g_ref, be_ref, o_ref, *, count):
    o_ref[...] = _bn(h_ref[...], s1_ref[...], s2_ref[...], g_ref[...],
                     be_ref[...], count)


def _bn_out(h, s1, s2, g, be, TM):
    BN, C = h.shape
    return pl.pallas_call(
        functools.partial(_bn_out_body, count=BN),
        grid=(BN // TM,),
        in_specs=[pl.BlockSpec((TM, C), lambda i: (i, 0)),
                  pl.BlockSpec((1, C), lambda i: (0, 0)),
                  pl.BlockSpec((1, C), lambda i: (0, 0)),
                  pl.BlockSpec((1, C), lambda i: (0, 0)),
                  pl.BlockSpec((1, C), lambda i: (0, 0))],
        out_specs=pl.BlockSpec((TM, C), lambda i: (i, 0)),
        out_shape=jax.ShapeDtypeStruct((BN, C), F32),
    )(h, s1, s2, g, be)


def _row2(v):
    return v.reshape(1, -1)


def _stage(x1p, x2tp, p1pm, src, W1aT, W1bT, b1, g1, be1, W2T, b2, g2, be2,
           TN, TM):
    B, N, _ = x1p.shape
    BN = B * N
    idx, w = _knn3(x1p, x2tp, TN)
    interp = _sc_interp(src, idx.reshape(BN * 3), w.reshape(BN * 3))
    h1, s1, s2 = _mm_stats(p1pm, W1aT, interp, W1bT, _row2(b1), TM)
    h2, t1, t2 = _bn_mm_stats(h1, s1, s2, _row2(g1), _row2(be1), W2T,
                              _row2(b2), TM)
    return _bn_out(h2, t1, t2, _row2(g2), _row2(be2), TM)


def _prep_x1(p):
    return jnp.pad(jnp.transpose(p, (0, 2, 1)), ((0, 0), (0, 0), (0, 5)))


def _prep_x2(p):
    return jnp.pad(p, ((0, 0), (0, 5), (0, 0)))


def _pm(feats):
    B, C, N = feats.shape
    return jnp.transpose(feats, (0, 2, 1)).reshape(B * N, C)


def kernel(pts0, pts1, pts2, pts3, feats0, feats1, feats2, feats3,
           fp3_W1, fp3_b1, fp3_g1, fp3_be1, fp3_W2, fp3_b2, fp3_g2, fp3_be2,
           fp2_W1, fp2_b1, fp2_g1, fp2_be1, fp2_W2, fp2_b2, fp2_g2, fp2_be2,
           fp1_W1, fp1_b1, fp1_g1, fp1_be1, fp1_W2, fp1_b2, fp1_g2, fp1_be2):
    B = pts0.shape[0]
    N0 = pts0.shape[2]

    l2 = _stage(_prep_x1(pts2), _prep_x2(pts3), _pm(feats2), _pm(feats3),
                fp3_W1[:, :512].T, fp3_W1[:, 512:].T, fp3_b1, fp3_g1, fp3_be1,
                fp3_W2.T, fp3_b2, fp3_g2, fp3_be2, TN=256, TM=1024)

    l1 = _stage(_prep_x1(pts1), _prep_x2(pts2), _pm(feats1), l2,
                fp2_W1[:, :320].T, fp2_W1[:, 320:].T, fp2_b1, fp2_g1, fp2_be1,
                fp2_W2.T, fp2_b2, fp2_g2, fp2_be2, TN=512, TM=2048)

    p1pm1 = jnp.pad(_pm(jnp.concatenate([pts0, feats0], axis=1)),
                    ((0, 0), (0, 2)))
    W1aT1 = jnp.pad(fp1_W1[:, :6].T, ((0, 2), (0, 0)))
    out = _stage(_prep_x1(pts0), _prep_x2(pts1), p1pm1, l1,
                 W1aT1, fp1_W1[:, 6:].T, fp1_b1, fp1_g1, fp1_be1,
                 fp1_W2.T, fp1_b2, fp1_g2, fp1_be2, TN=512, TM=2048)

    return jnp.transpose(out.reshape(B, N0, -1), (0, 2, 1))
